# baseline (speedup 1.0000x reference)
import jax
import jax.numpy as jnp
from jax.experimental import pallas as pl
from jax.experimental.pallas import tpu as pltpu

VOCAB = 32000
PAD = 0
SMOOTH = 0.1
CONF = 1.0 - SMOOTH
EPS = SMOOTH / (VOCAB - 2)

ROWS = 64


def _body(tgt_ref, x_ref, out_ref, acc_ref):
    i = pl.program_id(0)
    ni = pl.num_programs(0)
    x = x_ref[...]

    @pl.when(i == 0)
    def _init_acc():
        acc_ref[0] = 0.0

    m = jnp.max(x, axis=1, keepdims=True)
    s = jnp.sum(jnp.exp(x - m), axis=1, keepdims=True)
    sumx = jnp.sum(x, axis=1, keepdims=True)

    tgt = tgt_ref[...]
    col = jax.lax.broadcasted_iota(jnp.int32, (ROWS, VOCAB), 1)
    tv = jnp.sum(jnp.where(col == tgt, x, 0.0), axis=1, keepdims=True)

    lse = m + jnp.log(s)
    loss = lse - EPS * (sumx - x[:, 0:1]) - (CONF - EPS) * tv
    loss = jnp.where(tgt != PAD, loss, 0.0)
    acc_ref[0] += jnp.sum(loss)

    @pl.when(i == ni - 1)
    def _out():
        out_ref[0, 0] = acc_ref[0]


def kernel(pred, target):
    n = pred.shape[0] * pred.shape[1]
    pred2 = pred.reshape(n, VOCAB)
    ni = n // ROWS
    tgt = target.astype(jnp.int32).reshape(n, 1)

    out = pl.pallas_call(
        _body,
        grid=(ni,),
        in_specs=[
            pl.BlockSpec((ROWS, 1), lambda i: (i, 0)),
            pl.BlockSpec((ROWS, VOCAB), lambda i: (i, 0)),
        ],
        out_specs=pl.BlockSpec((1, 1), lambda i: (0, 0),
                               memory_space=pltpu.SMEM),
        out_shape=jax.ShapeDtypeStruct((1, 1), jnp.float32),
        scratch_shapes=[
            pltpu.SMEM((1,), jnp.float32),
        ],
        compiler_params=pltpu.CompilerParams(
            dimension_semantics=("arbitrary",)),
    )(tgt, pred2)
    return out[0, 0] / n
